# baseline (speedup 1.0000x reference)
import functools

import jax
import jax.numpy as jnp
from jax import lax
from jax.experimental import pallas as pl
from jax.experimental.pallas import tpu as pltpu
from jax.experimental.pallas import tpu_sc as plsc

_BATCH = 16384
_DIM = 64

_NC, _NS = 2, 16
_NW = _NC * _NS
_BPW = _BATCH // _NW


def _make_sc_gather():
    mesh = plsc.VectorSubcoreMesh(core_axis_name="c", subcore_axis_name="s")

    @functools.partial(
        pl.kernel,
        mesh=mesh,
        compiler_params=pltpu.CompilerParams(use_tc_tiling_on_sc=False),
        out_type=[
            jax.ShapeDtypeStruct((_BATCH, _DIM), jnp.float32),
            jax.ShapeDtypeStruct((_BATCH, _DIM), jnp.float32),
        ],
        scratch_types=[
            pltpu.VMEM((_BPW,), jnp.int32),
            pltpu.VMEM((_BPW,), jnp.int32),
            pltpu.VMEM((_BPW, _DIM), jnp.float32),
            pltpu.VMEM((_BPW, _DIM), jnp.float32),
            pltpu.SemaphoreType.DMA,
            pltpu.SemaphoreType.DMA,
        ],
    )
    def sc_gather(ent_hbm, rtb_hbm, aidx_hbm, ridx_hbm, a_out, r_out,
                  aidx_v, ridx_v, arows_v, rrows_v, sem_a, sem_r):
        wid = lax.axis_index("s") * _NC + lax.axis_index("c")
        base = wid * _BPW
        pltpu.sync_copy(aidx_hbm.at[pl.ds(base, _BPW)], aidx_v)
        pltpu.sync_copy(ridx_hbm.at[pl.ds(base, _BPW)], ridx_v)
        cp_a = pltpu.async_copy(ent_hbm.at[aidx_v], arows_v, sem_a)
        cp_r = pltpu.async_copy(rtb_hbm.at[ridx_v], rrows_v, sem_r)
        cp_a.wait()
        cp_r.wait()
        pltpu.sync_copy(arows_v, a_out.at[pl.ds(base, _BPW)])
        pltpu.sync_copy(rrows_v, r_out.at[pl.ds(base, _BPW)])

    return sc_gather


_sc_gather_cache = []


def _get_sc_gather():
    if not _sc_gather_cache:
        _sc_gather_cache.append(_make_sc_gather())
    return _sc_gather_cache[0]

_BLK = 1024


def _tc_body(a_ref, r_ref, w1t_ref, w2t_ref, b_ref, o_ref):
    acc = jnp.dot(a_ref[...], w1t_ref[...], preferred_element_type=jnp.float32)
    acc += jnp.dot(r_ref[...], w2t_ref[...], preferred_element_type=jnp.float32)
    o_ref[...] = jnp.maximum(acc + b_ref[...], 0.0)


@jax.jit
def _run(entity_table, relation_table, W, b, anchor, rel):
    a_rows, r_rows = _get_sc_gather()(entity_table, relation_table, anchor, rel)
    wt = W.T
    w1t = wt[:_DIM]
    w2t = wt[_DIM:]
    b2d = b.reshape(1, _DIM)
    out = pl.pallas_call(
        _tc_body,
        grid=(_BATCH // _BLK,),
        in_specs=[
            pl.BlockSpec((_BLK, _DIM), lambda i: (i, 0)),
            pl.BlockSpec((_BLK, _DIM), lambda i: (i, 0)),
            pl.BlockSpec((_DIM, _DIM), lambda i: (0, 0)),
            pl.BlockSpec((_DIM, _DIM), lambda i: (0, 0)),
            pl.BlockSpec((1, _DIM), lambda i: (0, 0)),
        ],
        out_specs=pl.BlockSpec((_BLK, _DIM), lambda i: (i, 0)),
        out_shape=jax.ShapeDtypeStruct((_BATCH, _DIM), jnp.float32),
    )(a_rows, r_rows, w1t, w2t, b2d)
    return out


def kernel(entity_table, relation_table, W, b, anchor, rel):
    return _run(entity_table, relation_table, W, b, anchor, rel)


# speedup vs baseline: 1.0906x; 1.0906x over previous
import jax
import jax.numpy as jnp
from jax import lax
from jax.experimental import pallas as pl
from jax.experimental.pallas import tpu as pltpu

_BATCH = 16384
_DIM = 64
_BLK = 512
_NBLK = _BATCH // _BLK
_RTB = 1000
_RTB_PAD = 1024


def _body(aidx_ref, ent_ref, rel_ref, rtb_ref, w1t_ref, w2t_ref, b_ref,
          o_ref, buf, sems):
    i = pl.program_id(0)

    def fire_block(j, s):
        def fire_one(k, carry):
            idx = aidx_ref[j * _BLK + k]
            pltpu.make_async_copy(
                ent_ref.at[pl.ds(idx, 1)],
                buf.at[s, pl.ds(k, 1)],
                sems.at[s],
            ).start()
            return carry

        lax.fori_loop(0, _BLK, fire_one, 0)

    def wait_block(s):
        def wait_one(k, carry):
            pltpu.make_async_copy(
                ent_ref.at[pl.ds(0, 1)],
                buf.at[s, pl.ds(0, 1)],
                sems.at[s],
            ).wait()
            return carry

        lax.fori_loop(0, _BLK, wait_one, 0)

    @pl.when(i == 0)
    def _():
        fire_block(0, 0)

    @pl.when(i + 1 < _NBLK)
    def _():
        fire_block(i + 1, (i + 1) % 2)

    wait_block(i % 2)

    a = buf[i % 2]
    rid = rel_ref[...]
    lanes = lax.broadcasted_iota(jnp.int32, (_BLK, _RTB_PAD), 1)
    onehot = jnp.where(lanes == rid, 1.0, 0.0).astype(jnp.float32)
    m2 = jnp.dot(rtb_ref[...], w2t_ref[...], preferred_element_type=jnp.float32)
    acc = jnp.dot(a, w1t_ref[...], preferred_element_type=jnp.float32)
    acc += jnp.dot(onehot, m2, preferred_element_type=jnp.float32)
    o_ref[...] = jnp.maximum(acc + b_ref[...], 0.0)


@jax.jit
def _run(entity_table, relation_table, W, b, anchor, rel):
    wt = W.T
    w1t = wt[:_DIM]
    w2t = wt[_DIM:]
    b2d = b.reshape(1, _DIM)
    r2d = rel.reshape(_BATCH, 1)
    rtb_pad = jnp.pad(relation_table, ((0, _RTB_PAD - _RTB), (0, 0)))
    grid_spec = pltpu.PrefetchScalarGridSpec(
        num_scalar_prefetch=1,
        grid=(_NBLK,),
        in_specs=[
            pl.BlockSpec(memory_space=pltpu.HBM),
            pl.BlockSpec((_BLK, 1), lambda i, aref: (i, 0)),
            pl.BlockSpec((_RTB_PAD, _DIM), lambda i, aref: (0, 0)),
            pl.BlockSpec((_DIM, _DIM), lambda i, aref: (0, 0)),
            pl.BlockSpec((_DIM, _DIM), lambda i, aref: (0, 0)),
            pl.BlockSpec((1, _DIM), lambda i, aref: (0, 0)),
        ],
        out_specs=pl.BlockSpec((_BLK, _DIM), lambda i, aref: (i, 0)),
        scratch_shapes=[
            pltpu.VMEM((2, _BLK, _DIM), jnp.float32),
            pltpu.SemaphoreType.DMA((2,)),
        ],
    )
    out = pl.pallas_call(
        _body,
        grid_spec=grid_spec,
        out_shape=jax.ShapeDtypeStruct((_BATCH, _DIM), jnp.float32),
        compiler_params=pltpu.CompilerParams(
            dimension_semantics=("arbitrary",),
        ),
    )(anchor, entity_table, r2d, rtb_pad, w1t, w2t, b2d)
    return out


def kernel(entity_table, relation_table, W, b, anchor, rel):
    return _run(entity_table, relation_table, W, b, anchor, rel)


# speedup vs baseline: 1.5145x; 1.3886x over previous
import jax
import jax.numpy as jnp
from jax import lax
from jax.experimental import pallas as pl
from jax.experimental.pallas import tpu as pltpu

_BATCH = 16384
_DIM = 64
_BLK = 512
_NBLK = _BATCH // _BLK
_RTB = 1000
_RTB_PAD = 1024


def _body(aidx_ref, ent_ref, rel_ref, rtb_ref, w1t_ref, w2t_ref, b_ref,
          o_ref, buf, sems):
    i = pl.program_id(0)

    def fire_block(j, s):
        def fire_four(k4, carry):
            for u in range(4):
                k = k4 * 4 + u
                idx = aidx_ref[j * _BLK + k]
                pltpu.make_async_copy(
                    ent_ref.at[pl.ds(idx, 1)],
                    buf.at[s, pl.ds(k, 1)],
                    sems.at[s],
                ).start()
            return carry

        lax.fori_loop(0, _BLK // 4, fire_four, 0)

    def wait_block(s):
        pltpu.make_async_copy(
            ent_ref.at[pl.ds(0, _BLK)],
            buf.at[s],
            sems.at[s],
        ).wait()

    @pl.when(i == 0)
    def _():
        fire_block(0, 0)

    @pl.when(i + 1 < _NBLK)
    def _():
        fire_block(i + 1, (i + 1) % 2)

    wait_block(i % 2)

    a = buf[i % 2]
    rid = rel_ref[...]
    lanes = lax.broadcasted_iota(jnp.int32, (_BLK, _RTB_PAD), 1)
    onehot = jnp.where(lanes == rid, 1.0, 0.0).astype(jnp.float32)
    m2 = jnp.dot(rtb_ref[...], w2t_ref[...], preferred_element_type=jnp.float32)
    acc = jnp.dot(a, w1t_ref[...], preferred_element_type=jnp.float32)
    acc += jnp.dot(onehot, m2, preferred_element_type=jnp.float32)
    o_ref[...] = jnp.maximum(acc + b_ref[...], 0.0)


@jax.jit
def _run(entity_table, relation_table, W, b, anchor, rel):
    wt = W.T
    w1t = wt[:_DIM]
    w2t = wt[_DIM:]
    b2d = b.reshape(1, _DIM)
    r2d = rel.reshape(_BATCH, 1)
    rtb_pad = jnp.pad(relation_table, ((0, _RTB_PAD - _RTB), (0, 0)))
    grid_spec = pltpu.PrefetchScalarGridSpec(
        num_scalar_prefetch=1,
        grid=(_NBLK,),
        in_specs=[
            pl.BlockSpec(memory_space=pltpu.HBM),
            pl.BlockSpec((_BLK, 1), lambda i, aref: (i, 0)),
            pl.BlockSpec((_RTB_PAD, _DIM), lambda i, aref: (0, 0)),
            pl.BlockSpec((_DIM, _DIM), lambda i, aref: (0, 0)),
            pl.BlockSpec((_DIM, _DIM), lambda i, aref: (0, 0)),
            pl.BlockSpec((1, _DIM), lambda i, aref: (0, 0)),
        ],
        out_specs=pl.BlockSpec((_BLK, _DIM), lambda i, aref: (i, 0)),
        scratch_shapes=[
            pltpu.VMEM((2, _BLK, _DIM), jnp.float32),
            pltpu.SemaphoreType.DMA((2,)),
        ],
    )
    out = pl.pallas_call(
        _body,
        grid_spec=grid_spec,
        out_shape=jax.ShapeDtypeStruct((_BATCH, _DIM), jnp.float32),
        compiler_params=pltpu.CompilerParams(
            dimension_semantics=("arbitrary",),
        ),
    )(anchor, entity_table, r2d, rtb_pad, w1t, w2t, b2d)
    return out


def kernel(entity_table, relation_table, W, b, anchor, rel):
    return _run(entity_table, relation_table, W, b, anchor, rel)


# speedup vs baseline: 1.5640x; 1.0326x over previous
import jax
import jax.numpy as jnp
from jax import lax
from jax.experimental import pallas as pl
from jax.experimental.pallas import tpu as pltpu

_BATCH = 16384
_DIM = 64
_BLK = 512
_NBLK = _BATCH // _BLK
_RTB = 1000
_RTB_PAD = 1024


def _body(aidx_ref, ent_ref, rel_ref, rtb_ref, w1t_ref, w2t_ref, b_ref,
          o_ref, buf, sems):
    i = pl.program_id(0)

    def fire_block(j, s):
        def fire_eight(k8, carry):
            k0 = k8 * 8
            idxs = [aidx_ref[j * _BLK + k0 + u] for u in range(8)]
            for u in range(8):
                pltpu.make_async_copy(
                    ent_ref.at[pl.ds(idxs[u], 1)],
                    buf.at[s, pl.ds(k0 + u, 1)],
                    sems.at[s],
                ).start()
            return carry

        lax.fori_loop(0, _BLK // 8, fire_eight, 0)

    def wait_block(s):
        pltpu.make_async_copy(
            ent_ref.at[pl.ds(0, _BLK)],
            buf.at[s],
            sems.at[s],
        ).wait()

    @pl.when(i == 0)
    def _():
        fire_block(0, 0)

    @pl.when(i + 1 < _NBLK)
    def _():
        fire_block(i + 1, (i + 1) % 2)

    wait_block(i % 2)

    a = buf[i % 2]
    rid = rel_ref[...]
    lanes = lax.broadcasted_iota(jnp.int32, (_BLK, _RTB_PAD), 1)
    onehot = jnp.where(lanes == rid, 1.0, 0.0).astype(jnp.float32)
    m2 = jnp.dot(rtb_ref[...], w2t_ref[...], preferred_element_type=jnp.float32)
    acc = jnp.dot(a, w1t_ref[...], preferred_element_type=jnp.float32)
    acc += jnp.dot(onehot, m2, preferred_element_type=jnp.float32)
    o_ref[...] = jnp.maximum(acc + b_ref[...], 0.0)


@jax.jit
def _run(entity_table, relation_table, W, b, anchor, rel):
    wt = W.T
    w1t = wt[:_DIM]
    w2t = wt[_DIM:]
    b2d = b.reshape(1, _DIM)
    r2d = rel.reshape(_BATCH, 1)
    rtb_pad = jnp.pad(relation_table, ((0, _RTB_PAD - _RTB), (0, 0)))
    grid_spec = pltpu.PrefetchScalarGridSpec(
        num_scalar_prefetch=1,
        grid=(_NBLK,),
        in_specs=[
            pl.BlockSpec(memory_space=pltpu.HBM),
            pl.BlockSpec((_BLK, 1), lambda i, aref: (i, 0)),
            pl.BlockSpec((_RTB_PAD, _DIM), lambda i, aref: (0, 0)),
            pl.BlockSpec((_DIM, _DIM), lambda i, aref: (0, 0)),
            pl.BlockSpec((_DIM, _DIM), lambda i, aref: (0, 0)),
            pl.BlockSpec((1, _DIM), lambda i, aref: (0, 0)),
        ],
        out_specs=pl.BlockSpec((_BLK, _DIM), lambda i, aref: (i, 0)),
        scratch_shapes=[
            pltpu.VMEM((2, _BLK, _DIM), jnp.float32),
            pltpu.SemaphoreType.DMA((2,)),
        ],
    )
    out = pl.pallas_call(
        _body,
        grid_spec=grid_spec,
        out_shape=jax.ShapeDtypeStruct((_BATCH, _DIM), jnp.float32),
        compiler_params=pltpu.CompilerParams(
            dimension_semantics=("arbitrary",),
        ),
    )(anchor, entity_table, r2d, rtb_pad, w1t, w2t, b2d)
    return out


def kernel(entity_table, relation_table, W, b, anchor, rel):
    return _run(entity_table, relation_table, W, b, anchor, rel)
